# Initial kernel scaffold; baseline (speedup 1.0000x reference)
import jax
import jax.numpy as jnp
from jax.experimental import pallas as pl


def kernel(input_encoded, timesteps):
    raise NotImplementedError("write your pallas kernel here")


# speedup vs baseline: 5.6897x; 5.6897x over previous
import functools

import numpy as np
import jax
import jax.numpy as jnp
from jax import lax
from jax.experimental import pallas as pl
from jax.experimental.pallas import tpu as pltpu
from jax.experimental.pallas import tpu_sc as plsc

_EMBED_DIM = 128
_MAX_LEN = 5000

_NC, _NS, _L = 2, 16, 16
_NW = _NC * _NS
_B, _S, _T, _D = 16, 8, 2048, _EMBED_DIM
_N = _B * _S * _T
_RPW = _N // _NW
_BATCH_ROWS = _S * _T
_C = 128
_NCHUNK = _RPW // _C


def _pos_table_np() -> np.ndarray:
    pos = np.arange(0, _MAX_LEN, dtype=np.float32)[:, None]
    factor = np.exp(
        np.arange(0, _EMBED_DIM, 2, dtype=np.float32)
        * (-np.log(10000.0) / _EMBED_DIM)
    )
    pe = np.zeros((_MAX_LEN, _EMBED_DIM), dtype=np.float32)
    pe[:, 0::2] = np.sin(pos * factor)
    pe[:, 1::2] = np.cos(pos * factor)
    return pe


_TABLE = _pos_table_np()


def _pe_body(x_hbm, ts_hbm, tab_hbm, out_hbm,
             ts_v, idx_v, inbuf, gbuf, obuf,
             sem_i0, sem_i1, sem_g0, sem_g1, sem_o0, sem_o1):
    sem_i = (sem_i0, sem_i1)
    sem_g = (sem_g0, sem_g1)
    sem_o = (sem_o0, sem_o1)

    wid = lax.axis_index("s") * _NC + lax.axis_index("c")
    base = wid * _RPW
    bstart = (wid // 2) * _BATCH_ROWS
    half = (wid % 2) * _RPW

    pltpu.sync_copy(ts_hbm.at[pl.ds(bstart, _BATCH_ROWS)], ts_v)

    def _min_body(i, m):
        for k in range(8):
            m = jnp.minimum(m, ts_v[pl.ds((i * 8 + k) * _L, _L)])
        return m

    m0 = ts_v[pl.ds(0, _L)]
    m = lax.fori_loop(0, _BATCH_ROWS // (8 * _L), _min_body, m0)
    mn = m[0]
    for i in range(1, _L):
        mn = jnp.minimum(mn, m[i])

    def _idx_body(j, carry):
        for k in range(_C // _L):
            v = ts_v[pl.ds(half + j * _C + k * _L, _L)]
            idx_v[j, pl.ds(k * _L, _L)] = v - mn
        return carry

    lax.fori_loop(0, _NCHUNK, _idx_body, 0)

    def _start_in(c, b):
        pltpu.async_copy(x_hbm.at[pl.ds(base + c * _C, _C)], inbuf.at[b], sem_i[b])

    def _start_gather(c, b):
        pltpu.async_copy(tab_hbm.at[idx_v.at[c]], gbuf.at[b], sem_g[b])

    def _start_out(c, b):
        pltpu.async_copy(obuf.at[b], out_hbm.at[pl.ds(base + c * _C, _C)], sem_o[b])

    def _wait_in(c, b):
        pltpu.make_async_copy(
            x_hbm.at[pl.ds(base + c * _C, _C)], inbuf.at[b], sem_i[b]).wait()

    def _wait_gather(c, b):
        pltpu.make_async_copy(
            tab_hbm.at[idx_v.at[c]], gbuf.at[b], sem_g[b]).wait()

    def _wait_out(c, b):
        pltpu.make_async_copy(
            obuf.at[b], out_hbm.at[pl.ds(base + c * _C, _C)], sem_o[b]).wait()

    for b in range(2):
        _start_in(b, b)
        _start_gather(b, b)

    @pl.loop(0, _NCHUNK, step=2)
    def _chunk_loop(i):
        for b in range(2):
            c = i + b
            _wait_in(c, b)
            _wait_gather(c, b)

            @pl.when(c >= 2)
            def _():
                _wait_out(c - 2, b)

            @pl.loop(0, _C)
            def _row_loop(r):
                for k in range(_D // _L):
                    sl = pl.ds(k * _L, _L)
                    obuf[b, r, sl] = inbuf[b, r, sl] + gbuf[b, r, sl]

            _start_out(c, b)

            @pl.when(c + 2 < _NCHUNK)
            def _():
                _start_in(c + 2, b)
                _start_gather(c + 2, b)

    for b in range(2):
        _wait_out(_NCHUNK - 2 + b, b)


@functools.partial(jax.jit, static_argnums=())
def _pe_call(x, ts, tab):
    mesh = plsc.VectorSubcoreMesh(core_axis_name="c", subcore_axis_name="s")
    f = pl.kernel(
        _pe_body,
        out_type=jax.ShapeDtypeStruct((_N, _D), jnp.float32),
        mesh=mesh,
        scratch_types=[
            pltpu.VMEM((_BATCH_ROWS,), jnp.int32),
            pltpu.VMEM((_NCHUNK, _C), jnp.int32),
            pltpu.VMEM((2, _C, _D), jnp.float32),
            pltpu.VMEM((2, _C, _D), jnp.float32),
            pltpu.VMEM((2, _C, _D), jnp.float32),
            pltpu.SemaphoreType.DMA,
            pltpu.SemaphoreType.DMA,
            pltpu.SemaphoreType.DMA,
            pltpu.SemaphoreType.DMA,
            pltpu.SemaphoreType.DMA,
            pltpu.SemaphoreType.DMA,
        ],
    )
    return f(x, ts, tab)


def kernel(input_encoded, timesteps):
    x = input_encoded.reshape(_N, _D)
    ts = timesteps.reshape(_N)
    tab = jnp.asarray(_TABLE)
    out = _pe_call(x, ts, tab)
    return out.reshape(input_encoded.shape)


# speedup vs baseline: 6.4034x; 1.1254x over previous
import functools

import numpy as np
import jax
import jax.numpy as jnp
from jax import lax
from jax.experimental import pallas as pl
from jax.experimental.pallas import tpu as pltpu
from jax.experimental.pallas import tpu_sc as plsc

_EMBED_DIM = 128
_MAX_LEN = 5000

_NC, _NS, _L = 2, 16, 16
_NW = _NC * _NS
_B, _S, _T, _D = 16, 8, 2048, _EMBED_DIM
_N = _B * _S * _T
_RPW = _N // _NW
_BATCH_ROWS = _S * _T
_C = 128
_NCHUNK = _RPW // _C


def _pos_table_np() -> np.ndarray:
    pos = np.arange(0, _MAX_LEN, dtype=np.float32)[:, None]
    factor = np.exp(
        np.arange(0, _EMBED_DIM, 2, dtype=np.float32)
        * (-np.log(10000.0) / _EMBED_DIM)
    )
    pe = np.zeros((_MAX_LEN, _EMBED_DIM), dtype=np.float32)
    pe[:, 0::2] = np.sin(pos * factor)
    pe[:, 1::2] = np.cos(pos * factor)
    return pe


_TABLE = _pos_table_np()


def _pe_body(x_hbm, ts_hbm, tab_hbm, out_hbm,
             ts_v, idx_v, tab_s, inbuf, gbuf,
             sem_i0, sem_i1, sem_g0, sem_g1, sem_o0, sem_o1, sem_t):
    sem_i = (sem_i0, sem_i1)
    sem_g = (sem_g0, sem_g1)
    sem_o = (sem_o0, sem_o1)

    sid = lax.axis_index("s")
    wid = sid * _NC + lax.axis_index("c")
    base = wid * _RPW
    bstart = (wid // 2) * _BATCH_ROWS
    half = (wid % 2) * _RPW

    pltpu.async_copy(x_hbm.at[pl.ds(base, _C)], inbuf.at[0], sem_i[0])
    pltpu.async_copy(x_hbm.at[pl.ds(base + _C, _C)], inbuf.at[1], sem_i[1])

    @pl.when(sid == 0)
    def _():
        pltpu.async_copy(tab_hbm, tab_s, sem_t)

    pltpu.sync_copy(ts_hbm.at[pl.ds(bstart, _BATCH_ROWS)], ts_v)

    def _min_body(i, m):
        for k in range(8):
            m = jnp.minimum(m, ts_v[pl.ds((i * 8 + k) * _L, _L)])
        return m

    m0 = ts_v[pl.ds(0, _L)]
    m = lax.fori_loop(0, _BATCH_ROWS // (8 * _L), _min_body, m0)
    mn = m[0]
    for i in range(1, _L):
        mn = jnp.minimum(mn, m[i])

    def _idx_body(j, carry):
        for k in range(_C // _L):
            v = ts_v[pl.ds(half + j * _C + k * _L, _L)]
            idx_v[j, pl.ds(k * _L, _L)] = v - mn
        return carry

    lax.fori_loop(0, _NCHUNK, _idx_body, 0)

    @pl.when(sid == 0)
    def _():
        pltpu.make_async_copy(tab_hbm, tab_s, sem_t).wait()
    plsc.subcore_barrier()

    def _start_in(c, b):
        pltpu.async_copy(x_hbm.at[pl.ds(base + c * _C, _C)], inbuf.at[b], sem_i[b])

    def _start_gather(c, b):
        pltpu.async_copy(tab_s.at[idx_v.at[c]], gbuf.at[b], sem_g[b])

    def _start_out(c, b):
        pltpu.async_copy(gbuf.at[b], out_hbm.at[pl.ds(base + c * _C, _C)], sem_o[b])

    def _wait_in(c, b):
        pltpu.make_async_copy(
            x_hbm.at[pl.ds(base + c * _C, _C)], inbuf.at[b], sem_i[b]).wait()

    def _wait_gather(c, b):
        pltpu.make_async_copy(
            tab_s.at[idx_v.at[c]], gbuf.at[b], sem_g[b]).wait()

    def _wait_out(c, b):
        pltpu.make_async_copy(
            gbuf.at[b], out_hbm.at[pl.ds(base + c * _C, _C)], sem_o[b]).wait()

    _start_gather(0, 0)

    @pl.loop(0, _NCHUNK, step=2)
    def _chunk_loop(i):
        for b in range(2):
            c = i + b
            _wait_in(c, b)
            _wait_gather(c, b)

            @pl.loop(0, _C)
            def _row_loop(r):
                for k in range(_D // _L):
                    sl = pl.ds(k * _L, _L)
                    gbuf[b, r, sl] = inbuf[b, r, sl] + gbuf[b, r, sl]

            _start_out(c, b)

            @pl.when(c + 2 < _NCHUNK)
            def _():
                _start_in(c + 2, b)

            @pl.when(c >= 1)
            def _():
                _wait_out(c - 1, 1 - b)

            @pl.when(c + 1 < _NCHUNK)
            def _():
                _start_gather(c + 1, 1 - b)

    _wait_out(_NCHUNK - 1, (_NCHUNK - 1) % 2)


@functools.partial(jax.jit, static_argnums=())
def _pe_call(x, ts, tab):
    mesh = plsc.VectorSubcoreMesh(core_axis_name="c", subcore_axis_name="s")
    f = pl.kernel(
        _pe_body,
        out_type=jax.ShapeDtypeStruct((_N, _D), jnp.float32),
        mesh=mesh,
        scratch_types=[
            pltpu.VMEM((_BATCH_ROWS,), jnp.int32),
            pltpu.VMEM((_NCHUNK, _C), jnp.int32),
            pltpu.VMEM_SHARED((_MAX_LEN, _D), jnp.float32),
            pltpu.VMEM((2, _C, _D), jnp.float32),
            pltpu.VMEM((2, _C, _D), jnp.float32),
            pltpu.SemaphoreType.DMA,
            pltpu.SemaphoreType.DMA,
            pltpu.SemaphoreType.DMA,
            pltpu.SemaphoreType.DMA,
            pltpu.SemaphoreType.DMA,
            pltpu.SemaphoreType.DMA,
            pltpu.SemaphoreType.DMA,
        ],
    )
    return f(x, ts, tab)


def kernel(input_encoded, timesteps):
    x = input_encoded.reshape(_N, _D)
    ts = timesteps.reshape(_N)
    tab = jnp.asarray(_TABLE)
    out = _pe_call(x, ts, tab)
    return out.reshape(input_encoded.shape)
